# Initial kernel scaffold; baseline (speedup 1.0000x reference)
import jax
import jax.numpy as jnp
from jax.experimental import pallas as pl


def kernel(x, pos, batch, x_skip, pos_skip, batch_skip, W1, b1, g1, be1, W2, b2, g2, be2):
    raise NotImplementedError("write your pallas kernel here")


# speedup vs baseline: 8.5625x; 8.5625x over previous
import functools

import jax
import jax.numpy as jnp
from jax import lax
from jax.experimental import pallas as pl
from jax.experimental.pallas import tpu as pltpu

N, M, B = 4096, 8192, 16
C_IN, C_SKIP = 256, 128
K = 3
H1, H2 = 512, 256
C_CAT = C_IN + C_SKIP

MB = 256
GRID1 = M // MB


def _leaky(h):
    return jnp.where(h >= 0, h, 0.01 * h)


def _k1_body(ps_ref, pt_ref, bs_ref, bc_ref, x_ref, xs_ref, w1a_ref, w1b_ref,
             b1_ref, z1_ref, s1_ref, q1_ref):
    qx = ps_ref[:, 0:1]
    qy = ps_ref[:, 1:2]
    qz = ps_ref[:, 2:3]
    px = pt_ref[0:1, :]
    py = pt_ref[1:2, :]
    pz = pt_ref[2:3, :]
    dx = qx - px
    dy = qy - py
    dz = qz - pz
    d2 = (dx * dx + dy * dy) + dz * dz
    same = bs_ref[:] == bc_ref[:]
    d2m = d2 + jnp.where(same, 0.0, 1e10)

    jcol = lax.broadcasted_iota(jnp.int32, (MB, N), 1)
    wrow = jnp.zeros((MB, N), jnp.float32)
    wsum = jnp.zeros((MB, 1), jnp.float32)
    cur = d2m
    for _ in range(K):
        m = jnp.min(cur, axis=1, keepdims=True)
        sel = cur == m
        idx = jnp.min(jnp.where(sel, jcol, N), axis=1, keepdims=True)
        chosen = jcol == idx
        val = jnp.sum(jnp.where(chosen, d2, 0.0), axis=1, keepdims=True)
        w = 1.0 / jnp.maximum(val, 1e-16)
        wrow = wrow + jnp.where(chosen, w, 0.0)
        wsum = wsum + w
        cur = jnp.where(chosen, jnp.float32(jnp.inf), cur)

    y = jnp.dot(wrow, x_ref[:], preferred_element_type=jnp.float32) / wsum
    z1 = (jnp.dot(y, w1a_ref[:], preferred_element_type=jnp.float32)
          + jnp.dot(xs_ref[:], w1b_ref[:], preferred_element_type=jnp.float32)
          + b1_ref[:])
    z1 = _leaky(z1)
    z1_ref[:] = z1

    @pl.when(pl.program_id(0) == 0)
    def _():
        s1_ref[:] = jnp.zeros_like(s1_ref)
        q1_ref[:] = jnp.zeros_like(q1_ref)

    s1_ref[:] += jnp.sum(z1, axis=0, keepdims=True)
    q1_ref[:] += jnp.sum(z1 * z1, axis=0, keepdims=True)


def _k2_body(z1_ref, a1_ref, c1_ref, w2_ref, b2_ref, z2_ref, s2_ref, q2_ref):
    h1 = z1_ref[:] * a1_ref[:] + c1_ref[:]
    z2 = jnp.dot(h1, w2_ref[:], preferred_element_type=jnp.float32) + b2_ref[:]
    z2 = _leaky(z2)
    z2_ref[:] = z2

    @pl.when(pl.program_id(0) == 0)
    def _():
        s2_ref[:] = jnp.zeros_like(s2_ref)
        q2_ref[:] = jnp.zeros_like(q2_ref)

    s2_ref[:] += jnp.sum(z2, axis=0, keepdims=True)
    q2_ref[:] += jnp.sum(z2 * z2, axis=0, keepdims=True)


def _k3_body(z2_ref, a2_ref, c2_ref, o_ref):
    o_ref[:] = z2_ref[:] * a2_ref[:] + c2_ref[:]


def _bn_affine(s, q, g, be):
    mu = s / M
    var = q / M - mu * mu
    a = g * lax.rsqrt(var + 1e-5)
    c = be - mu * a
    return a, c


@jax.jit
def kernel(x, pos, batch, x_skip, pos_skip, batch_skip,
           W1, b1, g1, be1, W2, b2, g2, be2):
    bs = batch_skip.astype(jnp.int32).reshape(M, 1)
    bc = batch.astype(jnp.int32).reshape(1, N)
    pos_t = pos.T
    w1a = W1[:, :C_IN].T
    w1b = W1[:, C_IN:].T
    w2t = W2.T

    z1, s1, q1 = pl.pallas_call(
        _k1_body,
        grid=(GRID1,),
        in_specs=[
            pl.BlockSpec((MB, 3), lambda i: (i, 0)),
            pl.BlockSpec((3, N), lambda i: (0, 0)),
            pl.BlockSpec((MB, 1), lambda i: (i, 0)),
            pl.BlockSpec((1, N), lambda i: (0, 0)),
            pl.BlockSpec((N, C_IN), lambda i: (0, 0)),
            pl.BlockSpec((MB, C_SKIP), lambda i: (i, 0)),
            pl.BlockSpec((C_IN, H1), lambda i: (0, 0)),
            pl.BlockSpec((C_SKIP, H1), lambda i: (0, 0)),
            pl.BlockSpec((1, H1), lambda i: (0, 0)),
        ],
        out_specs=[
            pl.BlockSpec((MB, H1), lambda i: (i, 0)),
            pl.BlockSpec((1, H1), lambda i: (0, 0)),
            pl.BlockSpec((1, H1), lambda i: (0, 0)),
        ],
        out_shape=[
            jax.ShapeDtypeStruct((M, H1), jnp.float32),
            jax.ShapeDtypeStruct((1, H1), jnp.float32),
            jax.ShapeDtypeStruct((1, H1), jnp.float32),
        ],
        compiler_params=pltpu.CompilerParams(
            dimension_semantics=("arbitrary",)),
    )(pos_skip, pos_t, bs, bc, x, x_skip, w1a, w1b, b1.reshape(1, H1))

    a1, c1 = _bn_affine(s1, q1, g1.reshape(1, H1), be1.reshape(1, H1))

    z2, s2, q2 = pl.pallas_call(
        _k2_body,
        grid=(GRID1,),
        in_specs=[
            pl.BlockSpec((MB, H1), lambda i: (i, 0)),
            pl.BlockSpec((1, H1), lambda i: (0, 0)),
            pl.BlockSpec((1, H1), lambda i: (0, 0)),
            pl.BlockSpec((H1, H2), lambda i: (0, 0)),
            pl.BlockSpec((1, H2), lambda i: (0, 0)),
        ],
        out_specs=[
            pl.BlockSpec((MB, H2), lambda i: (i, 0)),
            pl.BlockSpec((1, H2), lambda i: (0, 0)),
            pl.BlockSpec((1, H2), lambda i: (0, 0)),
        ],
        out_shape=[
            jax.ShapeDtypeStruct((M, H2), jnp.float32),
            jax.ShapeDtypeStruct((1, H2), jnp.float32),
            jax.ShapeDtypeStruct((1, H2), jnp.float32),
        ],
        compiler_params=pltpu.CompilerParams(
            dimension_semantics=("arbitrary",)),
    )(z1, a1, c1, w2t, b2.reshape(1, H2))

    a2, c2 = _bn_affine(s2, q2, g2.reshape(1, H2), be2.reshape(1, H2))

    MB3 = 1024
    h = pl.pallas_call(
        _k3_body,
        grid=(M // MB3,),
        in_specs=[
            pl.BlockSpec((MB3, H2), lambda i: (i, 0)),
            pl.BlockSpec((1, H2), lambda i: (0, 0)),
            pl.BlockSpec((1, H2), lambda i: (0, 0)),
        ],
        out_specs=pl.BlockSpec((MB3, H2), lambda i: (i, 0)),
        out_shape=jax.ShapeDtypeStruct((M, H2), jnp.float32),
    )(z2, a2, c2)

    return (h, pos_skip, batch_skip)
